# baseline (speedup 1.0000x reference)
import functools

import jax
import jax.numpy as jnp
from jax import lax
from jax.experimental import pallas as pl
from jax.experimental.pallas import tpu as pltpu
from jax.experimental.pallas import tpu_sc as plsc

_ND = 20000
_NDP = 20480
_DPW = _NDP // 16
_DVR = _DPW // 16
_NO = 1000
_NOP = 1024
_OPW = _NOP // 32
_DET_PAD_LBL = 21
_OBJ_PAD_LBL = 31


def _k1_body(dl_hbm, order_out, cnt_out,
             lbl_v, ord_v, cnt_v, off_v, sem):
    wid = lax.axis_index("s")
    base = wid * _DPW
    pltpu.sync_copy(dl_hbm.at[pl.ds(base, _DPW)], lbl_v)

    iota = jnp.arange(16, dtype=jnp.int32)
    zero16 = jnp.zeros((16,), jnp.int32)

    cnt_v[pl.ds(0, 16)] = zero16
    cnt_v[pl.ds(16, 16)] = zero16

    def hist_body(v, _):
        lv = lbl_v[pl.ds(v * 16, 16)]
        rank, lastm = plsc.scan_count(lv)
        cur = plsc.load_gather(cnt_v, [lv])
        plsc.store_scatter(cnt_v, [lv], cur + rank, mask=lastm)
        return 0

    lax.fori_loop(0, _DVR, hist_body, 0)

    row0 = cnt_v[pl.ds(0, 16)]
    row1 = cnt_v[pl.ds(16, 16)]
    ex0 = plsc.cumsum(row0) - row0
    ex1 = plsc.cumsum(row1) - row1 + jnp.sum(row0)
    off_v[pl.ds(0, 16)] = ex0
    off_v[pl.ds(16, 16)] = ex1

    def scat_body(v, _):
        lv = lbl_v[pl.ds(v * 16, 16)]
        rank, lastm = plsc.scan_count(lv)
        offs = plsc.load_gather(off_v, [lv])
        pos = offs + rank - 1
        plsc.store_scatter(ord_v, [pos], base + iota + v * 16)
        plsc.store_scatter(off_v, [lv], offs + rank, mask=lastm)
        return 0

    lax.fori_loop(0, _DVR, scat_body, 0)

    cp1 = pltpu.async_copy(ord_v, order_out.at[pl.ds(base, _DPW)], sem)
    cp2 = pltpu.async_copy(cnt_v, cnt_out.at[pl.ds(wid * 32, 32)], sem)
    cp1.wait()
    cp2.wait()


def _k2_body(dx_hbm, dy_hbm, dX_hbm, dY_hbm, order_hbm, cnt_hbm,
             gx_hbm, gy_hbm, gX_hbm, gY_hbm, gl_hbm,
             part_out,
             px_v, py_v, pX_v, pY_v, ord_v, cnt_v, lex_v,
             gx_v, gy_v, gX_v, gY_v, gl_v, m_v, acc_v, sem):
    wid = lax.axis_index("s") * 2 + lax.axis_index("c")
    base = wid * _OPW
    cps = [pltpu.async_copy(dx_hbm, px_v, sem),
           pltpu.async_copy(dy_hbm, py_v, sem),
           pltpu.async_copy(dX_hbm, pX_v, sem),
           pltpu.async_copy(dY_hbm, pY_v, sem),
           pltpu.async_copy(order_hbm, ord_v, sem),
           pltpu.async_copy(cnt_hbm, cnt_v, sem),
           pltpu.async_copy(gx_hbm.at[pl.ds(base, _OPW)],
                            gx_v.at[pl.ds(0, _OPW)], sem),
           pltpu.async_copy(gy_hbm.at[pl.ds(base, _OPW)],
                            gy_v.at[pl.ds(0, _OPW)], sem),
           pltpu.async_copy(gX_hbm.at[pl.ds(base, _OPW)],
                            gX_v.at[pl.ds(0, _OPW)], sem),
           pltpu.async_copy(gY_hbm.at[pl.ds(base, _OPW)],
                            gY_v.at[pl.ds(0, _OPW)], sem),
           pltpu.async_copy(gl_hbm.at[pl.ds(base, _OPW)],
                            gl_v.at[pl.ds(0, _OPW)], sem)]
    for cp in cps:
        cp.wait()

    iota = jnp.arange(16, dtype=jnp.int32)
    iotaf = iota.astype(jnp.float32)
    acc_v[...] = jnp.zeros((16,), jnp.float32)

    def lex_body(w, _):
        row0 = cnt_v[pl.ds(w * 32, 16)]
        row1 = cnt_v[pl.ds(w * 32 + 16, 16)]
        lex_v[pl.ds(w * 32, 16)] = plsc.cumsum(row0) - row0
        lex_v[pl.ds(w * 32 + 16, 16)] = (
            plsc.cumsum(row1) - row1 + jnp.sum(row0))
        return 0

    lax.fori_loop(0, 16, lex_body, 0)

    def obj_body(j, carry):
        c = gl_v[pl.ds(j, 16)][0]
        x0s = jnp.full((16,), gx_v[pl.ds(j, 16)][0], jnp.float32)
        y0s = jnp.full((16,), gy_v[pl.ds(j, 16)][0], jnp.float32)
        x1s = jnp.full((16,), gX_v[pl.ds(j, 16)][0], jnp.float32)
        y1s = jnp.full((16,), gY_v[pl.ds(j, 16)][0], jnp.float32)
        ag = (x1s - x0s) * (y1s - y0s)
        m_v[...] = jnp.full((16,), -1.0, jnp.float32)

        def run_body(w, _):
            s = w * _DPW + lex_v[pl.ds(w * 32 + c, 16)][0]
            e = s + cnt_v[pl.ds(w * 32 + c, 16)][0]
            s0 = (s >> 4) << 4
            nv = ((e + 15) >> 4) - (s >> 4)

            def vr_body(t, _):
                p0 = pl.multiple_of(s0 + t * 16, 16)
                idxv = ord_v[pl.ds(p0, 16)]
                xd = plsc.load_gather(px_v, [idxv])
                yd = plsc.load_gather(py_v, [idxv])
                Xd = plsc.load_gather(pX_v, [idxv])
                Yd = plsc.load_gather(pY_v, [idxv])
                ad = (Xd - xd) * (Yd - yd)
                lox = jnp.maximum(xd, x0s)
                loy = jnp.maximum(yd, y0s)
                hix = jnp.minimum(Xd, x1s)
                hiy = jnp.minimum(Yd, y1s)
                iw = jnp.maximum(hix - lox, 0.0)
                ih = jnp.maximum(hiy - loy, 0.0)
                inter = iw * ih
                union = (ag + ad) - inter
                iou = inter / union
                pos = p0 + iota
                ok = (pos >= s) & (pos < e)
                m_v[...] = jnp.maximum(m_v[...], jnp.where(ok, iou, -1.0))
                return 0

            lax.fori_loop(0, nv, vr_body, 0)
            return 0

        lax.fori_loop(0, 16, run_body, 0)
        best = jnp.max(m_v[...])
        matched = best >= 0.0
        contrib = jnp.where(matched, 1.0 - best, 0.0)
        cntc = jnp.where(matched, 1.0, 0.0)
        acc_v[...] = acc_v[...] + jnp.where(iotaf == 0.0, contrib, 0.0) \
            + jnp.where(iotaf == 1.0, cntc, 0.0)
        return carry

    lax.fori_loop(0, _OPW, obj_body, 0)
    pltpu.sync_copy(acc_v, part_out.at[wid])


def _k3_reduce(part_ref, out_ref):
    p = part_ref[...]
    s = jnp.sum(p[:, 0:1], keepdims=True)
    n = jnp.sum(p[:, 1:2], keepdims=True)
    out_ref[...] = (s / n).reshape(1, 1)


def kernel(det_boxes, det_scores, det_labels, boxes, labels):
    del det_scores
    f32, i32 = jnp.float32, jnp.int32
    db = det_boxes[0].astype(f32)
    dl = det_labels[0].astype(i32)
    gb = boxes[0].astype(f32)
    gl = labels[0].astype(i32)

    dx = jnp.zeros((_NDP,), f32).at[:_ND].set(db[:, 0])
    dy = jnp.zeros((_NDP,), f32).at[:_ND].set(db[:, 1])
    dX = jnp.zeros((_NDP,), f32).at[:_ND].set(db[:, 2])
    dY = jnp.zeros((_NDP,), f32).at[:_ND].set(db[:, 3])
    dlp = jnp.full((_NDP,), _DET_PAD_LBL, i32).at[:_ND].set(dl)

    gx = jnp.zeros((_NOP,), f32).at[:_NO].set(gb[:, 0])
    gy = jnp.zeros((_NOP,), f32).at[:_NO].set(gb[:, 1])
    gX = jnp.zeros((_NOP,), f32).at[:_NO].set(gb[:, 2])
    gY = jnp.zeros((_NOP,), f32).at[:_NO].set(gb[:, 3])
    glp = jnp.full((_NOP,), _OBJ_PAD_LBL, i32).at[:_NO].set(gl)

    mesh1 = plsc.VectorSubcoreMesh(
        core_axis_name="c", subcore_axis_name="s", num_cores=1,
        num_subcores=16)
    k1 = functools.partial(
        pl.kernel,
        out_type=(jax.ShapeDtypeStruct((_NDP,), i32),
                  jax.ShapeDtypeStruct((512,), i32)),
        mesh=mesh1,
        compiler_params=pltpu.CompilerParams(needs_layout_passes=False),
        scratch_types=[
            pltpu.VMEM((_DPW,), i32),
            pltpu.VMEM((_DPW,), i32),
            pltpu.VMEM((32,), i32),
            pltpu.VMEM((32,), i32),
            pltpu.SemaphoreType.DMA,
        ])(_k1_body)
    order, cnts = k1(dlp)

    mesh2 = plsc.VectorSubcoreMesh(
        core_axis_name="c", subcore_axis_name="s", num_cores=2,
        num_subcores=16)
    k2 = functools.partial(
        pl.kernel,
        out_type=jax.ShapeDtypeStruct((32, 16), f32),
        mesh=mesh2,
        compiler_params=pltpu.CompilerParams(needs_layout_passes=False),
        scratch_types=[
            pltpu.VMEM((_NDP,), f32), pltpu.VMEM((_NDP,), f32),
            pltpu.VMEM((_NDP,), f32), pltpu.VMEM((_NDP,), f32),
            pltpu.VMEM((_NDP,), i32),
            pltpu.VMEM((512,), i32),
            pltpu.VMEM((512,), i32),
            pltpu.VMEM((_OPW + 16,), f32), pltpu.VMEM((_OPW + 16,), f32),
            pltpu.VMEM((_OPW + 16,), f32), pltpu.VMEM((_OPW + 16,), f32),
            pltpu.VMEM((_OPW + 16,), i32),
            pltpu.VMEM((16,), f32),
            pltpu.VMEM((16,), f32),
            pltpu.SemaphoreType.DMA,
        ])(_k2_body)
    parts = k2(dx, dy, dX, dY, order, cnts, gx, gy, gX, gY, glp)

    out = pl.pallas_call(
        _k3_reduce,
        out_shape=jax.ShapeDtypeStruct((1, 1), f32),
    )(parts)
    return out[0, 0]
